# baseline (speedup 1.0000x reference)
import jax
import jax.numpy as jnp
from jax import lax
from jax.experimental import pallas as pl
from jax.experimental.pallas import tpu as pltpu

_K = 8192
_D = 32
_T = 512
_KC = 2048


def _vq_tile_kernel(x_ref, s1_ref, cb_ref, s2_ref, codes_ref, q_ref, part_ref):
    x = x_ref[...]
    s1 = s1_ref[...]
    s2 = s2_ref[...]

    best_val = None
    best_idx = None
    best_q = None
    num_chunks = _K // _KC
    for c in range(num_chunks):
        cb_c = cb_ref[c * _KC:(c + 1) * _KC, :]
        s2_c = s2[:, c * _KC:(c + 1) * _KC]
        mm = lax.dot_general(
            x, cb_c, (((1,), (1,)), ((), ())),
            preferred_element_type=jnp.float32,
        )
        d = (s1 + s2_c) - 2.0 * mm
        local_min = jnp.min(d, axis=1, keepdims=True)
        iota = lax.broadcasted_iota(jnp.int32, (_T, _KC), 1)
        local_idx = jnp.min(
            jnp.where(d == local_min, iota, _K), axis=1, keepdims=True
        )
        oh = (iota == local_idx).astype(jnp.float32)
        q_c = lax.dot_general(
            oh, cb_c, (((1,), (0,)), ((), ())),
            preferred_element_type=jnp.float32,
            precision=lax.Precision.HIGHEST,
        )
        if c == 0:
            best_val = local_min
            best_idx = local_idx
            best_q = q_c
        else:
            upd = local_min < best_val
            best_val = jnp.where(upd, local_min, best_val)
            best_idx = jnp.where(upd, local_idx + c * _KC, best_idx)
            best_q = jnp.where(upd, q_c, best_q)

    codes_ref[...] = best_idx
    q_ref[...] = x + (best_q - x)
    err = best_q - x
    part_ref[0] = jnp.sum(err * err, axis=(0, 1), keepdims=True)


def kernel(x, codebook):
    B, S, D = x.shape
    N = B * S
    s1 = jnp.sum(x ** 2, axis=-1, keepdims=True)
    s2 = jnp.sum(codebook ** 2, axis=-1)

    x2 = x.reshape(N, D)
    s1_2 = s1.reshape(N, 1)
    s2_2 = s2.reshape(1, _K)
    grid = (N // _T,)

    codes2, q2, parts = pl.pallas_call(
        _vq_tile_kernel,
        grid=grid,
        in_specs=[
            pl.BlockSpec((_T, D), lambda i: (i, 0)),
            pl.BlockSpec((_T, 1), lambda i: (i, 0)),
            pl.BlockSpec((_K, D), lambda i: (0, 0)),
            pl.BlockSpec((1, _K), lambda i: (0, 0)),
        ],
        out_specs=[
            pl.BlockSpec((_T, 1), lambda i: (i, 0)),
            pl.BlockSpec((_T, D), lambda i: (i, 0)),
            pl.BlockSpec((1, 1, 1), lambda i: (i, 0, 0)),
        ],
        out_shape=[
            jax.ShapeDtypeStruct((N, 1), jnp.int32),
            jax.ShapeDtypeStruct((N, D), jnp.float32),
            jax.ShapeDtypeStruct((grid[0], 1, 1), jnp.float32),
        ],
        compiler_params=pltpu.CompilerParams(
            dimension_semantics=("parallel",),
        ),
    )(x2, s1_2, codebook, s2_2)

    codes = codes2.reshape(B, S)
    quantized_st = q2.reshape(B, S, D)
    loss = 2.0 * (jnp.sum(parts) / (N * D))
    return (quantized_st, codes, loss)


# speedup vs baseline: 3.0406x; 3.0406x over previous
import functools

import jax
import jax.numpy as jnp
from jax import lax
from jax.experimental import pallas as pl
from jax.experimental.pallas import tpu as pltpu
from jax.experimental.pallas import tpu_sc as plsc

_K = 8192
_D = 32
_T = 512
_KC = 2048


def _vq_tile_kernel(x_ref, s1_ref, cb_ref, s2_ref, codes_ref, part_ref):
    x = x_ref[...]
    s1 = s1_ref[...]
    s2 = s2_ref[...]

    best_val = None
    best_idx = None
    num_chunks = _K // _KC
    for c in range(num_chunks):
        cb_c = cb_ref[c * _KC:(c + 1) * _KC, :]
        s2_c = s2[:, c * _KC:(c + 1) * _KC]
        mm = lax.dot_general(
            x, cb_c, (((1,), (1,)), ((), ())),
            preferred_element_type=jnp.float32,
        )
        d = (s1 + s2_c) - 2.0 * mm
        local_min = jnp.min(d, axis=1, keepdims=True)
        iota = lax.broadcasted_iota(jnp.int32, (_T, _KC), 1)
        local_idx = jnp.min(
            jnp.where(d == local_min, iota, _K), axis=1, keepdims=True
        )
        if c == 0:
            best_val = local_min
            best_idx = local_idx
        else:
            upd = local_min < best_val
            best_val = jnp.where(upd, local_min, best_val)
            best_idx = jnp.where(upd, local_idx + c * _KC, best_idx)

    codes_ref[...] = best_idx
    part_ref[0] = jnp.sum(best_val, axis=(0, 1), keepdims=True)


def _make_sc_gather(V, DP, B):
    info = plsc.get_sparse_core_info()
    NC, NS = info.num_cores, info.num_subcores
    NW = NC * NS
    b_per_w = B // NW
    CH = 256
    n_ch = b_per_w // CH
    mesh = plsc.VectorSubcoreMesh(core_axis_name="c", subcore_axis_name="s")

    @functools.partial(
        pl.kernel, mesh=mesh,
        out_type=jax.ShapeDtypeStruct((B, DP), jnp.float32),
        scratch_types=[
            pltpu.VMEM((b_per_w,), jnp.int32),
            pltpu.VMEM((CH, DP), jnp.float32),
            pltpu.SemaphoreType.DMA,
        ],
    )
    def sc_gather(table_hbm, idx_hbm, out_hbm, idx_v, rows_v, sem):
        wid = lax.axis_index("s") * NC + lax.axis_index("c")
        base = wid * b_per_w
        pltpu.sync_copy(idx_hbm.at[pl.ds(base, b_per_w)], idx_v)
        for cc in range(n_ch):
            pltpu.async_copy(
                table_hbm.at[idx_v.at[pl.ds(cc * CH, CH)]], rows_v, sem
            ).wait()
            pltpu.sync_copy(rows_v, out_hbm.at[pl.ds(base + cc * CH, CH)])

    return sc_gather


def kernel(x, codebook):
    B, S, D = x.shape
    N = B * S
    s1 = jnp.sum(x ** 2, axis=-1, keepdims=True)
    s2 = jnp.sum(codebook ** 2, axis=-1)

    x2 = x.reshape(N, D)
    s1_2 = s1.reshape(N, 1)
    s2_2 = s2.reshape(1, _K)
    grid = (N // _T,)

    codes2, parts = pl.pallas_call(
        _vq_tile_kernel,
        grid=grid,
        in_specs=[
            pl.BlockSpec((_T, D), lambda i: (i, 0)),
            pl.BlockSpec((_T, 1), lambda i: (i, 0)),
            pl.BlockSpec((_K, D), lambda i: (0, 0)),
            pl.BlockSpec((1, _K), lambda i: (0, 0)),
        ],
        out_specs=[
            pl.BlockSpec((_T, 1), lambda i: (i, 0)),
            pl.BlockSpec((1, 1, 1), lambda i: (i, 0, 0)),
        ],
        out_shape=[
            jax.ShapeDtypeStruct((N, 1), jnp.int32),
            jax.ShapeDtypeStruct((grid[0], 1, 1), jnp.float32),
        ],
        compiler_params=pltpu.CompilerParams(
            dimension_semantics=("parallel",),
        ),
    )(x2, s1_2, codebook, s2_2)

    codes_flat = codes2.reshape(N)
    cb_pad = jnp.pad(codebook, ((0, 0), (0, 128 - D)))
    q_pad = _make_sc_gather(_K, 128, N)(cb_pad, codes_flat)
    q2 = q_pad[:, :D]

    codes = codes2.reshape(B, S)
    quantized_st = q2.reshape(B, S, D)
    loss = 2.0 * (jnp.sum(parts) / (N * D))
    return (quantized_st, codes, loss)


# speedup vs baseline: 3.3738x; 1.1096x over previous
import functools

import jax
import jax.numpy as jnp
from jax import lax
from jax.experimental import pallas as pl
from jax.experimental.pallas import tpu as pltpu
from jax.experimental.pallas import tpu_sc as plsc

_K = 8192
_D = 32
_T = 512
_KC = 2048


def _vq_tile_kernel(xs_ref, s1_ref, cb_ref, s2_ref, codes_ref, part_ref):
    xs = xs_ref[...]
    s1 = s1_ref[...]
    s2 = s2_ref[...]

    best_val = None
    best_idx = None
    num_chunks = _K // _KC
    for c in range(num_chunks):
        cb_c = cb_ref[c * _KC:(c + 1) * _KC, :]
        s2_c = s2[:, c * _KC:(c + 1) * _KC]
        mm2 = lax.dot_general(
            xs, cb_c, (((1,), (1,)), ((), ())),
            preferred_element_type=jnp.float32,
        )
        d = (s1 + s2_c) - mm2
        local_min = jnp.min(d, axis=1, keepdims=True)
        fiota = lax.broadcasted_iota(jnp.int32, (_T, _KC), 1).astype(jnp.float32)
        local_idx = jnp.min(
            jnp.where(d == local_min, fiota, float(_K)), axis=1, keepdims=True
        )
        if c == 0:
            best_val = local_min
            best_idx = local_idx
        else:
            upd = local_min < best_val
            best_val = jnp.where(upd, local_min, best_val)
            best_idx = jnp.where(upd, local_idx + float(c * _KC), best_idx)

    codes_ref[...] = best_idx.astype(jnp.int32)
    part_ref[0] = jnp.sum(best_val, axis=(0, 1), keepdims=True)


def _make_sc_gather(V, DP, B):
    info = plsc.get_sparse_core_info()
    NC, NS = info.num_cores, info.num_subcores
    NW = NC * NS
    b_per_w = B // NW
    CH = 256
    n_ch = b_per_w // CH
    mesh = plsc.VectorSubcoreMesh(core_axis_name="c", subcore_axis_name="s")

    @functools.partial(
        pl.kernel, mesh=mesh,
        out_type=jax.ShapeDtypeStruct((B, DP), jnp.float32),
        scratch_types=[
            pltpu.VMEM((b_per_w,), jnp.int32),
            pltpu.VMEM((CH, DP), jnp.float32),
            pltpu.SemaphoreType.DMA,
        ],
    )
    def sc_gather(table_hbm, idx_hbm, out_hbm, idx_v, rows_v, sem):
        wid = lax.axis_index("s") * NC + lax.axis_index("c")
        base = wid * b_per_w
        pltpu.sync_copy(idx_hbm.at[pl.ds(base, b_per_w)], idx_v)
        for cc in range(n_ch):
            pltpu.async_copy(
                table_hbm.at[idx_v.at[pl.ds(cc * CH, CH)]], rows_v, sem
            ).wait()
            pltpu.sync_copy(rows_v, out_hbm.at[pl.ds(base + cc * CH, CH)])

    return sc_gather


def kernel(x, codebook):
    B, S, D = x.shape
    N = B * S
    s1 = jnp.sum(x ** 2, axis=-1, keepdims=True)
    s2 = jnp.sum(codebook ** 2, axis=-1)

    x2 = x.reshape(N, D)
    s1_2 = s1.reshape(N, 1)
    s2_2 = s2.reshape(1, _K)
    grid = (N // _T,)

    codes2, parts = pl.pallas_call(
        _vq_tile_kernel,
        grid=grid,
        in_specs=[
            pl.BlockSpec((_T, D), lambda i: (i, 0)),
            pl.BlockSpec((_T, 1), lambda i: (i, 0)),
            pl.BlockSpec((_K, D), lambda i: (0, 0)),
            pl.BlockSpec((1, _K), lambda i: (0, 0)),
        ],
        out_specs=[
            pl.BlockSpec((_T, 1), lambda i: (i, 0)),
            pl.BlockSpec((1, 1, 1), lambda i: (i, 0, 0)),
        ],
        out_shape=[
            jax.ShapeDtypeStruct((N, 1), jnp.int32),
            jax.ShapeDtypeStruct((grid[0], 1, 1), jnp.float32),
        ],
        compiler_params=pltpu.CompilerParams(
            dimension_semantics=("parallel",),
        ),
    )(x2 * 2.0, s1_2, codebook, s2_2)

    codes_flat = codes2.reshape(N)
    cb_pad = jnp.pad(codebook, ((0, 0), (0, 128 - D)))
    q_pad = _make_sc_gather(_K, 128, N)(cb_pad, codes_flat)
    q2 = q_pad[:, :D]

    codes = codes2.reshape(B, S)
    quantized_st = q2.reshape(B, S, D)
    loss = 2.0 * (jnp.sum(parts) / (N * D))
    return (quantized_st, codes, loss)


# speedup vs baseline: 4.1450x; 1.2286x over previous
import functools

import jax
import jax.numpy as jnp
from jax import lax
from jax.experimental import pallas as pl
from jax.experimental.pallas import tpu as pltpu
from jax.experimental.pallas import tpu_sc as plsc

_K = 8192
_D = 32
_T = 512
_KC = 2048
_RB = 64


def _vq_tile_kernel(xs_ref, s1_ref, cb_ref, s2_ref, codes_ref, part_ref):
    xs = xs_ref[...]
    s1 = s1_ref[...]
    s2 = s2_ref[...]

    num_chunks = _K // _KC
    mm2 = []
    for c in range(num_chunks):
        cb_c = cb_ref[c * _KC:(c + 1) * _KC, :]
        mm2.append(lax.dot_general(
            xs, cb_c, (((1,), (1,)), ((), ())),
            preferred_element_type=jnp.float32,
        ))

    num_h = _K // 128
    h_per_chunk = _KC // 128
    part = None
    for rb in range(_T // _RB):
        r0 = rb * _RB
        s1_r = s1[r0:r0 + _RB, :]
        m_acc = None
        h_acc = None
        for h in range(num_h):
            c, j = divmod(h, h_per_chunk)
            sl = mm2[c][r0:r0 + _RB, j * 128:(j + 1) * 128]
            s2_h = s2[:, h * 128:(h + 1) * 128]
            dj = (s1_r + s2_h) - sl
            if h == 0:
                m_acc = dj
                h_acc = jnp.zeros((_RB, 128), jnp.float32)
            else:
                mask = dj < m_acc
                h_acc = jnp.where(mask, jnp.float32(h), h_acc)
                m_acc = jnp.minimum(m_acc, dj)
        gmin = jnp.min(m_acc, axis=1, keepdims=True)
        liota = lax.broadcasted_iota(jnp.int32, (_RB, 128), 1).astype(jnp.float32)
        code_f = jnp.min(
            jnp.where(m_acc == gmin, h_acc * 128.0 + liota, jnp.float32(_K)),
            axis=1, keepdims=True,
        )
        codes_ref[r0:r0 + _RB, :] = code_f.astype(jnp.int32)
        psum = jnp.sum(gmin, axis=(0, 1), keepdims=True)
        part = psum if part is None else part + psum
    part_ref[0] = part


def _make_sc_gather(V, DP, B):
    info = plsc.get_sparse_core_info()
    NC, NS = info.num_cores, info.num_subcores
    NW = NC * NS
    b_per_w = B // NW
    CH = 256
    n_ch = b_per_w // CH
    mesh = plsc.VectorSubcoreMesh(core_axis_name="c", subcore_axis_name="s")

    @functools.partial(
        pl.kernel, mesh=mesh,
        out_type=jax.ShapeDtypeStruct((B, DP), jnp.float32),
        scratch_types=[
            pltpu.VMEM((b_per_w,), jnp.int32),
            pltpu.VMEM((CH, DP), jnp.float32),
            pltpu.SemaphoreType.DMA,
        ],
    )
    def sc_gather(table_hbm, idx_hbm, out_hbm, idx_v, rows_v, sem):
        wid = lax.axis_index("s") * NC + lax.axis_index("c")
        base = wid * b_per_w
        pltpu.sync_copy(idx_hbm.at[pl.ds(base, b_per_w)], idx_v)
        for cc in range(n_ch):
            pltpu.async_copy(
                table_hbm.at[idx_v.at[pl.ds(cc * CH, CH)]], rows_v, sem
            ).wait()
            pltpu.sync_copy(rows_v, out_hbm.at[pl.ds(base + cc * CH, CH)])

    return sc_gather


def kernel(x, codebook):
    B, S, D = x.shape
    N = B * S
    s1 = jnp.sum(x ** 2, axis=-1, keepdims=True)
    s2 = jnp.sum(codebook ** 2, axis=-1)

    x2 = x.reshape(N, D)
    s1_2 = s1.reshape(N, 1)
    s2_2 = s2.reshape(1, _K)
    grid = (N // _T,)

    codes2, parts = pl.pallas_call(
        _vq_tile_kernel,
        grid=grid,
        in_specs=[
            pl.BlockSpec((_T, D), lambda i: (i, 0)),
            pl.BlockSpec((_T, 1), lambda i: (i, 0)),
            pl.BlockSpec((_K, D), lambda i: (0, 0)),
            pl.BlockSpec((1, _K), lambda i: (0, 0)),
        ],
        out_specs=[
            pl.BlockSpec((_T, 1), lambda i: (i, 0)),
            pl.BlockSpec((1, 1, 1), lambda i: (i, 0, 0)),
        ],
        out_shape=[
            jax.ShapeDtypeStruct((N, 1), jnp.int32),
            jax.ShapeDtypeStruct((grid[0], 1, 1), jnp.float32),
        ],
        compiler_params=pltpu.CompilerParams(
            dimension_semantics=("parallel",),
        ),
    )(x2 * 2.0, s1_2, codebook, s2_2)

    codes_flat = codes2.reshape(N)
    cb_pad = jnp.pad(codebook, ((0, 0), (0, 128 - D)))
    q_pad = _make_sc_gather(_K, 128, N)(cb_pad, codes_flat)
    q2 = q_pad[:, :D]

    codes = codes2.reshape(B, S)
    quantized_st = q2.reshape(B, S, D)
    loss = 2.0 * (jnp.sum(parts) / (N * D))
    return (quantized_st, codes, loss)


# speedup vs baseline: 4.4257x; 1.0677x over previous
import functools

import jax
import jax.numpy as jnp
from jax import lax
from jax.experimental import pallas as pl
from jax.experimental.pallas import tpu as pltpu
from jax.experimental.pallas import tpu_sc as plsc

_K = 8192
_D = 32
_T = 1024
_KC = 2048
_RB = 64


def _vq_tile_kernel(xs_ref, s1_ref, cb_ref, s2_ref, codes_ref, part_ref):
    xs = xs_ref[...]
    s1 = s1_ref[...]
    s2 = s2_ref[...]

    num_chunks = _K // _KC
    mm2 = []
    for c in range(num_chunks):
        cb_c = cb_ref[c * _KC:(c + 1) * _KC, :]
        mm2.append(lax.dot_general(
            xs, cb_c, (((1,), (1,)), ((), ())),
            preferred_element_type=jnp.float32,
        ))

    num_h = _K // 128
    h_per_chunk = _KC // 128
    part = None
    for rb in range(_T // _RB):
        r0 = rb * _RB
        s1_r = s1[r0:r0 + _RB, :]
        m_acc = None
        h_acc = None
        for h in range(num_h):
            c, j = divmod(h, h_per_chunk)
            sl = mm2[c][r0:r0 + _RB, j * 128:(j + 1) * 128]
            s2_h = s2[:, h * 128:(h + 1) * 128]
            dj = (s1_r + s2_h) - sl
            if h == 0:
                m_acc = dj
                h_acc = jnp.zeros((_RB, 128), jnp.float32)
            else:
                mask = dj < m_acc
                h_acc = jnp.where(mask, jnp.float32(h), h_acc)
                m_acc = jnp.minimum(m_acc, dj)
        gmin = jnp.min(m_acc, axis=1, keepdims=True)
        liota = lax.broadcasted_iota(jnp.int32, (_RB, 128), 1).astype(jnp.float32)
        code_f = jnp.min(
            jnp.where(m_acc == gmin, h_acc * 128.0 + liota, jnp.float32(_K)),
            axis=1, keepdims=True,
        )
        codes_ref[r0:r0 + _RB, :] = code_f.astype(jnp.int32)
        psum = jnp.sum(gmin, axis=(0, 1), keepdims=True)
        part = psum if part is None else part + psum
    part_ref[0] = part


def _make_sc_gather(V, DP, B):
    info = plsc.get_sparse_core_info()
    NC, NS = info.num_cores, info.num_subcores
    NW = NC * NS
    b_per_w = B // NW
    CH = 256
    n_ch = b_per_w // CH
    mesh = plsc.VectorSubcoreMesh(core_axis_name="c", subcore_axis_name="s")

    @functools.partial(
        pl.kernel, mesh=mesh,
        out_type=jax.ShapeDtypeStruct((B, DP), jnp.float32),
        scratch_types=[
            pltpu.VMEM((b_per_w,), jnp.int32),
            pltpu.VMEM((CH, DP), jnp.float32),
            pltpu.SemaphoreType.DMA,
        ],
    )
    def sc_gather(table_hbm, idx_hbm, out_hbm, idx_v, rows_v, sem):
        wid = lax.axis_index("s") * NC + lax.axis_index("c")
        base = wid * b_per_w
        pltpu.sync_copy(idx_hbm.at[pl.ds(base, b_per_w)], idx_v)
        for cc in range(n_ch):
            pltpu.async_copy(
                table_hbm.at[idx_v.at[pl.ds(cc * CH, CH)]], rows_v, sem
            ).wait()
            pltpu.sync_copy(rows_v, out_hbm.at[pl.ds(base + cc * CH, CH)])

    return sc_gather


def kernel(x, codebook):
    B, S, D = x.shape
    N = B * S
    s1 = jnp.sum(x ** 2, axis=-1, keepdims=True)
    s2 = jnp.sum(codebook ** 2, axis=-1)

    x2 = x.reshape(N, D)
    s1_2 = s1.reshape(N, 1)
    s2_2 = s2.reshape(1, _K)
    grid = (N // _T,)

    codes2, parts = pl.pallas_call(
        _vq_tile_kernel,
        grid=grid,
        in_specs=[
            pl.BlockSpec((_T, D), lambda i: (i, 0)),
            pl.BlockSpec((_T, 1), lambda i: (i, 0)),
            pl.BlockSpec((_K, D), lambda i: (0, 0)),
            pl.BlockSpec((1, _K), lambda i: (0, 0)),
        ],
        out_specs=[
            pl.BlockSpec((_T, 1), lambda i: (i, 0)),
            pl.BlockSpec((1, 1, 1), lambda i: (i, 0, 0)),
        ],
        out_shape=[
            jax.ShapeDtypeStruct((N, 1), jnp.int32),
            jax.ShapeDtypeStruct((grid[0], 1, 1), jnp.float32),
        ],
        compiler_params=pltpu.CompilerParams(
            dimension_semantics=("parallel",),
        ),
    )(x2 * 2.0, s1_2, codebook, s2_2)

    codes_flat = codes2.reshape(N)
    cb_pad = jnp.pad(codebook, ((0, 0), (0, 128 - D)))
    q_pad = _make_sc_gather(_K, 128, N)(cb_pad, codes_flat)
    q2 = q_pad[:, :D]

    codes = codes2.reshape(B, S)
    quantized_st = q2.reshape(B, S, D)
    loss = 2.0 * (jnp.sum(parts) / (N * D))
    return (quantized_st, codes, loss)


# speedup vs baseline: 4.5071x; 1.0184x over previous
import functools

import jax
import jax.numpy as jnp
from jax import lax
from jax.experimental import pallas as pl
from jax.experimental.pallas import tpu as pltpu
from jax.experimental.pallas import tpu_sc as plsc

_K = 8192
_D = 32
_T = 1024
_KC = 2048
_RB = 64


def _vq_tile_kernel(xs_ref, s1_ref, cb_ref, s2_ref, codes_ref, part_ref):
    xs = xs_ref[...] * 2.0
    s1 = s1_ref[...]
    s2 = s2_ref[...]

    num_chunks = _K // _KC
    mm2 = []
    for c in range(num_chunks):
        cb_c = cb_ref[c * _KC:(c + 1) * _KC, :]
        mm2.append(lax.dot_general(
            xs, cb_c, (((1,), (1,)), ((), ())),
            preferred_element_type=jnp.float32,
        ))

    num_h = _K // 128
    h_per_chunk = _KC // 128
    part = None
    for rb in range(_T // _RB):
        r0 = rb * _RB
        s1_r = s1[r0:r0 + _RB, :]
        m_acc = None
        h_acc = None
        for h in range(num_h):
            c, j = divmod(h, h_per_chunk)
            sl = mm2[c][r0:r0 + _RB, j * 128:(j + 1) * 128]
            s2_h = s2[:, h * 128:(h + 1) * 128]
            dj = (s1_r + s2_h) - sl
            if h == 0:
                m_acc = dj
                h_acc = jnp.zeros((_RB, 128), jnp.float32)
            else:
                mask = dj < m_acc
                h_acc = jnp.where(mask, jnp.float32(h), h_acc)
                m_acc = jnp.minimum(m_acc, dj)
        gmin = jnp.min(m_acc, axis=1, keepdims=True)
        liota = lax.broadcasted_iota(jnp.int32, (_RB, 128), 1).astype(jnp.float32)
        code_f = jnp.min(
            jnp.where(m_acc == gmin, h_acc * 128.0 + liota, jnp.float32(_K)),
            axis=1, keepdims=True,
        )
        codes_ref[r0:r0 + _RB, :] = code_f.astype(jnp.int32)
        psum = jnp.sum(gmin, axis=(0, 1), keepdims=True)
        part = psum if part is None else part + psum
    part_ref[0] = part


def _make_sc_gather(V, DP, B):
    info = plsc.get_sparse_core_info()
    NC, NS = info.num_cores, info.num_subcores
    NW = NC * NS
    b_per_w = B // NW
    CH = 256
    n_ch = b_per_w // CH
    mesh = plsc.VectorSubcoreMesh(core_axis_name="c", subcore_axis_name="s")

    @functools.partial(
        pl.kernel, mesh=mesh,
        out_type=jax.ShapeDtypeStruct((B, DP), jnp.float32),
        scratch_types=[
            pltpu.VMEM((b_per_w,), jnp.int32),
            pltpu.VMEM((CH, DP), jnp.float32),
            pltpu.SemaphoreType.DMA,
        ],
    )
    def sc_gather(table_hbm, idx_hbm, out_hbm, idx_v, rows_v, sem):
        wid = lax.axis_index("s") * NC + lax.axis_index("c")
        base = wid * b_per_w
        pltpu.sync_copy(idx_hbm.at[pl.ds(base, b_per_w)], idx_v)
        for cc in range(n_ch):
            pltpu.async_copy(
                table_hbm.at[idx_v.at[pl.ds(cc * CH, CH)]], rows_v, sem
            ).wait()
            pltpu.sync_copy(rows_v, out_hbm.at[pl.ds(base + cc * CH, CH)])

    return sc_gather


def kernel(x, codebook):
    B, S, D = x.shape
    N = B * S
    s1 = jnp.sum(x ** 2, axis=-1, keepdims=True)
    s2 = jnp.sum(codebook ** 2, axis=-1)

    x2 = x.reshape(N, D)
    s1_2 = s1.reshape(N, 1)
    s2_2 = s2.reshape(1, _K)
    grid = (N // _T,)

    codes2, parts = pl.pallas_call(
        _vq_tile_kernel,
        grid=grid,
        in_specs=[
            pl.BlockSpec((_T, D), lambda i: (i, 0)),
            pl.BlockSpec((_T, 1), lambda i: (i, 0)),
            pl.BlockSpec((_K, D), lambda i: (0, 0)),
            pl.BlockSpec((1, _K), lambda i: (0, 0)),
        ],
        out_specs=[
            pl.BlockSpec((_T, 1), lambda i: (i, 0)),
            pl.BlockSpec((1, 1, 1), lambda i: (i, 0, 0)),
        ],
        out_shape=[
            jax.ShapeDtypeStruct((N, 1), jnp.int32),
            jax.ShapeDtypeStruct((grid[0], 1, 1), jnp.float32),
        ],
        compiler_params=pltpu.CompilerParams(
            dimension_semantics=("parallel",),
        ),
    )(x2, s1_2, codebook, s2_2)

    codes_flat = codes2.reshape(N)
    cb_pad = jnp.pad(codebook, ((0, 0), (0, 128 - D)))
    q_pad = _make_sc_gather(_K, 128, N)(cb_pad, codes_flat)
    q2 = q_pad[:, :D]

    codes = codes2.reshape(B, S)
    quantized_st = q2.reshape(B, S, D)
    loss = 2.0 * (jnp.sum(parts) / (N * D))
    return (quantized_st, codes, loss)


# speedup vs baseline: 4.5844x; 1.0172x over previous
import functools

import jax
import jax.numpy as jnp
from jax import lax
from jax.experimental import pallas as pl
from jax.experimental.pallas import tpu as pltpu
from jax.experimental.pallas import tpu_sc as plsc

_K = 8192
_D = 32
_T = 1024
_KC = 2048
_RB = 64


def _vq_tile_kernel(xs_ref, cb_ref, s2_ref, codes_ref, part_ref):
    xv = xs_ref[...]
    xs = xv * 2.0
    s1 = jnp.sum(xv * xv, axis=1, keepdims=True)
    s2 = s2_ref[...]

    num_chunks = _K // _KC
    mm2 = []
    for c in range(num_chunks):
        cb_c = cb_ref[c * _KC:(c + 1) * _KC, :]
        mm2.append(lax.dot_general(
            xs, cb_c, (((1,), (1,)), ((), ())),
            preferred_element_type=jnp.float32,
        ))

    num_h = _K // 128
    h_per_chunk = _KC // 128
    part = None
    for rb in range(_T // _RB):
        r0 = rb * _RB
        s1_r = s1[r0:r0 + _RB, :]
        m_acc = None
        h_acc = None
        for h in range(num_h):
            c, j = divmod(h, h_per_chunk)
            sl = mm2[c][r0:r0 + _RB, j * 128:(j + 1) * 128]
            s2_h = s2[:, h * 128:(h + 1) * 128]
            dj = (s1_r + s2_h) - sl
            if h == 0:
                m_acc = dj
                h_acc = jnp.zeros((_RB, 128), jnp.float32)
            else:
                mask = dj < m_acc
                h_acc = jnp.where(mask, jnp.float32(h), h_acc)
                m_acc = jnp.minimum(m_acc, dj)
        gmin = jnp.min(m_acc, axis=1, keepdims=True)
        liota = lax.broadcasted_iota(jnp.int32, (_RB, 128), 1).astype(jnp.float32)
        code_f = jnp.min(
            jnp.where(m_acc == gmin, h_acc * 128.0 + liota, jnp.float32(_K)),
            axis=1, keepdims=True,
        )
        codes_ref[r0:r0 + _RB, :] = code_f.astype(jnp.int32)
        psum = jnp.sum(gmin, axis=(0, 1), keepdims=True)
        part = psum if part is None else part + psum
    part_ref[0] = part


def _make_sc_gather(V, DP, B):
    info = plsc.get_sparse_core_info()
    NC, NS = info.num_cores, info.num_subcores
    NW = NC * NS
    b_per_w = B // NW
    CH = 256
    n_ch = b_per_w // CH
    mesh = plsc.VectorSubcoreMesh(core_axis_name="c", subcore_axis_name="s")

    @functools.partial(
        pl.kernel, mesh=mesh,
        out_type=jax.ShapeDtypeStruct((B, DP), jnp.float32),
        scratch_types=[
            pltpu.VMEM((b_per_w,), jnp.int32),
            pltpu.VMEM((CH, DP), jnp.float32),
            pltpu.SemaphoreType.DMA,
        ],
    )
    def sc_gather(table_hbm, idx_hbm, out_hbm, idx_v, rows_v, sem):
        wid = lax.axis_index("s") * NC + lax.axis_index("c")
        base = wid * b_per_w
        pltpu.sync_copy(idx_hbm.at[pl.ds(base, b_per_w)], idx_v)
        for cc in range(n_ch):
            pltpu.async_copy(
                table_hbm.at[idx_v.at[pl.ds(cc * CH, CH)]], rows_v, sem
            ).wait()
            pltpu.sync_copy(rows_v, out_hbm.at[pl.ds(base + cc * CH, CH)])

    return sc_gather


def kernel(x, codebook):
    B, S, D = x.shape
    N = B * S
    s2 = jnp.sum(codebook ** 2, axis=-1)

    x2 = x.reshape(N, D)
    s2_2 = s2.reshape(1, _K)
    grid = (N // _T,)

    codes2, parts = pl.pallas_call(
        _vq_tile_kernel,
        grid=grid,
        in_specs=[
            pl.BlockSpec((_T, D), lambda i: (i, 0)),
            pl.BlockSpec((_K, D), lambda i: (0, 0)),
            pl.BlockSpec((1, _K), lambda i: (0, 0)),
        ],
        out_specs=[
            pl.BlockSpec((_T, 1), lambda i: (i, 0)),
            pl.BlockSpec((1, 1, 1), lambda i: (i, 0, 0)),
        ],
        out_shape=[
            jax.ShapeDtypeStruct((N, 1), jnp.int32),
            jax.ShapeDtypeStruct((grid[0], 1, 1), jnp.float32),
        ],
        compiler_params=pltpu.CompilerParams(
            dimension_semantics=("parallel",),
        ),
    )(x2, codebook, s2_2)

    codes_flat = codes2.reshape(N)
    cb_pad = jnp.pad(codebook, ((0, 0), (0, 128 - D)))
    q_pad = _make_sc_gather(_K, 128, N)(cb_pad, codes_flat)
    q2 = q_pad[:, :D]

    codes = codes2.reshape(B, S)
    quantized_st = q2.reshape(B, S, D)
    loss = 2.0 * (jnp.sum(parts) / (N * D))
    return (quantized_st, codes, loss)
